# baseline (speedup 1.0000x reference)
import functools

import jax
import jax.numpy as jnp
from jax import lax
from jax.experimental import pallas as pl
from jax.experimental.pallas import tpu as pltpu
from jax.experimental.pallas import tpu_sc as plsc

_SHIFT = 8192
_B, _CH, _LEN = 64, 2, 160000
_OUT = _LEN - _SHIFT
_ROWS = _B * _CH
_NW = 32
_RPW = _ROWS // _NW
_NT = _OUT // 128
_CHUNKS = [180] * 6 + [106]
_MMAX = max(_CHUNKS)
_UNROLL = 8
_NIB = 3

_OFFS = [
    5316, 4114, 1207, 7361, 653, 7531, 2433, 2343, 6150, 5378, 552, 6130,
    7577, 475, 8140, 1810, 5707, 4994, 2883, 519, 3638, 651, 2316, 7875,
    3180, 1553, 7152, 539, 6428, 3383, 6405, 676, 1493, 2094, 3123, 2068,
    4910, 6066, 3921, 6125, 5895, 5700, 3735, 381, 7033, 4288, 3388, 6820,
    4899, 5645, 5780, 7899, 978, 371, 2040, 439, 2059, 5458, 1883, 3001,
    6937, 7942, 1824, 3720]


@functools.partial(
    pl.kernel,
    out_type=jax.ShapeDtypeStruct((_B, _CH, _OUT), jnp.float32),
    mesh=plsc.VectorSubcoreMesh(core_axis_name="c", subcore_axis_name="s"),
    scratch_types=[
        *[pltpu.VMEM(((_MMAX + 1) * 128,), jnp.float32) for _ in range(_NIB)],
        *[pltpu.VMEM((_MMAX * 128,), jnp.float32) for _ in range(2)],
        *[pltpu.SemaphoreType.DMA for _ in range(_NIB + 2)],
    ],
    compiler_params=pltpu.CompilerParams(use_tc_tiling_on_sc=True),
)
def _shift_copy(wav_hbm, out_hbm, in0, in1, in2, out0, out1, si0, si1, si2, so0, so1):
    wid = lax.axis_index("s") * 2 + lax.axis_index("c")
    ins, outs = (in0, in1, in2), (out0, out1)
    sin, sout = (si0, si1, si2), (so0, so1)

    def batch_off(kb):
        acc = jnp.int32(_OFFS[kb])
        for w in range(1, _NW):
            acc = jnp.where(wid == w, jnp.int32(_OFFS[w * 2 + kb]), acc)
        return acc

    boffs = [batch_off(0), batch_off(1)]

    tasks = []
    for k in range(_RPW):
        lt0 = 0
        for m in _CHUNKS:
            tasks.append((k, lt0, m))
            lt0 += m
    _T = len(tasks)

    def params(t):
        k, lt0, m = tasks[t]
        off = boffs[k // _CH]
        r128 = lax.bitwise_and(off, 127)
        t_al = pl.multiple_of(off - r128, 128)
        b = wid * (_RPW // _CH) + k // _CH
        c = k % _CH
        return b, c, t_al + lt0 * 128, r128, lt0 * 128, m

    def start_in(t):
        b, c, t_in0, _, _, m = params(t)
        return pltpu.async_copy(
            wav_hbm.at[b, c, pl.ds(t_in0, (m + 1) * 128)],
            ins[t % _NIB].at[pl.ds(0, (m + 1) * 128)], sin[t % _NIB])

    in_h, out_h = {}, {}
    for t in range(_NIB):
        in_h[t] = start_in(t)
    for t in range(_T):
        b, c, _, r128, dst0, m = params(t)
        in_h[t].wait()
        if t >= 2:
            out_h[t - 2].wait()
        ib, ob = ins[t % _NIB], outs[t % 2]

        @plsc.parallel_loop(0, m * 8, step=_UNROLL)
        def _(i):
            for u in range(_UNROLL):
                ob[pl.ds((i + u) * 16, 16)] = ib[pl.ds((i + u) * 16 + r128, 16)]

        out_h[t] = pltpu.async_copy(
            ob.at[pl.ds(0, m * 128)],
            out_hbm.at[b, c, pl.ds(dst0, m * 128)], sout[t % 2])
        if t + _NIB < _T:
            in_h[t + _NIB] = start_in(t + _NIB)
    out_h[_T - 2].wait()
    out_h[_T - 1].wait()


def kernel(raw_wav):
    return _shift_copy(raw_wav)


# speedup vs baseline: 1.0162x; 1.0162x over previous
import functools

import jax
import jax.numpy as jnp
from jax import lax
from jax.experimental import pallas as pl
from jax.experimental.pallas import tpu as pltpu
from jax.experimental.pallas import tpu_sc as plsc

_SHIFT = 8192
_B, _CH, _LEN = 64, 2, 160000
_OUT = _LEN - _SHIFT
_ROWS = _B * _CH
_NW = 32
_RPW = _ROWS // _NW
_NT = _OUT // 128
_CHUNKS = [238, 238, 238, 238, 234]
_MMAX = max(_CHUNKS)
_UNROLL = 8
_NIB = 2

_OFFS = [
    5316, 4114, 1207, 7361, 653, 7531, 2433, 2343, 6150, 5378, 552, 6130,
    7577, 475, 8140, 1810, 5707, 4994, 2883, 519, 3638, 651, 2316, 7875,
    3180, 1553, 7152, 539, 6428, 3383, 6405, 676, 1493, 2094, 3123, 2068,
    4910, 6066, 3921, 6125, 5895, 5700, 3735, 381, 7033, 4288, 3388, 6820,
    4899, 5645, 5780, 7899, 978, 371, 2040, 439, 2059, 5458, 1883, 3001,
    6937, 7942, 1824, 3720]


@functools.partial(
    pl.kernel,
    out_type=jax.ShapeDtypeStruct((_B, _CH, _OUT), jnp.float32),
    mesh=plsc.VectorSubcoreMesh(core_axis_name="c", subcore_axis_name="s"),
    scratch_types=[
        *[pltpu.VMEM(((_MMAX + 1) * 128,), jnp.float32) for _ in range(_NIB)],
        *[pltpu.VMEM((_MMAX * 128,), jnp.float32) for _ in range(2)],
        *[pltpu.SemaphoreType.DMA for _ in range(_NIB + 2)],
    ],
    compiler_params=pltpu.CompilerParams(use_tc_tiling_on_sc=True),
)
def _shift_copy(wav_hbm, out_hbm, in0, in1, out0, out1, si0, si1, so0, so1):
    wid = lax.axis_index("s") * 2 + lax.axis_index("c")
    ins, outs = (in0, in1), (out0, out1)
    sin, sout = (si0, si1), (so0, so1)

    def batch_off(kb):
        acc = jnp.int32(_OFFS[kb])
        for w in range(1, _NW):
            acc = jnp.where(wid == w, jnp.int32(_OFFS[w * 2 + kb]), acc)
        return acc

    boffs = [batch_off(0), batch_off(1)]

    tasks = []
    for k in range(_RPW):
        lt0 = 0
        for m in _CHUNKS:
            tasks.append((k, lt0, m))
            lt0 += m
    _T = len(tasks)

    def params(t):
        k, lt0, m = tasks[t]
        off = boffs[k // _CH]
        r128 = lax.bitwise_and(off, 127)
        t_al = pl.multiple_of(off - r128, 128)
        b = wid * (_RPW // _CH) + k // _CH
        c = k % _CH
        return b, c, t_al + lt0 * 128, r128, lt0 * 128, m

    def start_in(t):
        b, c, t_in0, _, _, m = params(t)
        return pltpu.async_copy(
            wav_hbm.at[b, c, pl.ds(t_in0, (m + 1) * 128)],
            ins[t % _NIB].at[pl.ds(0, (m + 1) * 128)], sin[t % _NIB])

    in_h, out_h = {}, {}
    for t in range(_NIB):
        in_h[t] = start_in(t)
    for t in range(_T):
        b, c, _, r128, dst0, m = params(t)
        in_h[t].wait()
        if t >= 2:
            out_h[t - 2].wait()
        ib, ob = ins[t % _NIB], outs[t % 2]

        @plsc.parallel_loop(0, m * 8, step=_UNROLL)
        def _(i):
            for u in range(_UNROLL):
                ob[pl.ds((i + u) * 16, 16)] = ib[pl.ds((i + u) * 16 + r128, 16)]

        out_h[t] = pltpu.async_copy(
            ob.at[pl.ds(0, m * 128)],
            out_hbm.at[b, c, pl.ds(dst0, m * 128)], sout[t % 2])
        if t + _NIB < _T:
            in_h[t + _NIB] = start_in(t + _NIB)
    out_h[_T - 2].wait()
    out_h[_T - 1].wait()


def kernel(raw_wav):
    return _shift_copy(raw_wav)
